# Initial kernel scaffold; baseline (speedup 1.0000x reference)
import jax
import jax.numpy as jnp
from jax.experimental import pallas as pl


def kernel(efeat, nfeat, flow_features, edge_index, W1, b1, g1, be1, W2, b2, g2, be2, W3, b3):
    raise NotImplementedError("write your pallas kernel here")


# speedup vs baseline: 3.5655x; 3.5655x over previous
import functools

import jax
import jax.numpy as jnp
from jax import lax
from jax.experimental import pallas as pl
from jax.experimental.pallas import tpu as pltpu
from jax.experimental.pallas import tpu_sc as plsc

N = 10000
E = 320000
D_N = 128
D_E = 128
D_F = 16
H = 256
D_OUT = 128

NC = 2
NS = 16
CHUNK = 80
ROWS_PER_TILE = 624
TAIL_ROWS = N - NS * ROWS_PER_TILE


EPW = E // (NC * NS)
NCHUNK = EPW // CHUNK


def _segment_sum_sc(efeat, dst, zeros):
    mesh = plsc.VectorSubcoreMesh(
        core_axis_name="c", subcore_axis_name="s", num_cores=NC, num_subcores=NS
    )

    @functools.partial(
        pl.kernel,
        out_type=jax.ShapeDtypeStruct((NC, N, D_E), jnp.float32),
        mesh=mesh,
        scratch_types=[
            pltpu.VMEM_SHARED((N, D_E), jnp.float32),
            pltpu.VMEM((CHUNK, D_E), jnp.float32),
            pltpu.VMEM((CHUNK, D_E), jnp.float32),
            pltpu.VMEM((CHUNK,), jnp.int32),
            pltpu.VMEM((CHUNK,), jnp.int32),
            pltpu.SemaphoreType.DMA,
            pltpu.SemaphoreType.DMA,
            pltpu.SemaphoreType.DMA,
            pltpu.SemaphoreType.DMA,
        ],
    )
    def k(efeat_hbm, dst_hbm, z_hbm, out_hbm, acc,
          rows0, rows1, idx0, idx1, sr0, sr1, si0, si1):
        cid = lax.axis_index("c")
        sid = lax.axis_index("s")
        r0 = sid * ROWS_PER_TILE
        pltpu.sync_copy(
            z_hbm.at[pl.ds(r0, ROWS_PER_TILE)], acc.at[pl.ds(r0, ROWS_PER_TILE)]
        )

        @pl.when(sid == NS - 1)
        def _():
            pltpu.sync_copy(
                z_hbm.at[pl.ds(NS * ROWS_PER_TILE, TAIL_ROWS)],
                acc.at[pl.ds(NS * ROWS_PER_TILE, TAIL_ROWS)],
            )

        plsc.subcore_barrier()

        eb = (cid * NS + sid) * EPW

        def start(i, rows, idx, sr, si):
            b = eb + i * CHUNK
            pltpu.async_copy(efeat_hbm.at[pl.ds(b, CHUNK)], rows, sr)
            pltpu.async_copy(dst_hbm.at[pl.ds(b, CHUNK)], idx, si)

        def wait(rows, idx, sr, si):
            pltpu.make_async_copy(efeat_hbm.at[pl.ds(0, CHUNK)], rows, sr).wait()
            pltpu.make_async_copy(dst_hbm.at[pl.ds(0, CHUNK)], idx, si).wait()

        def scatter_add(rows, idx):
            pltpu.sync_copy(rows, acc.at[idx], add=True)

        start(0, rows0, idx0, sr0, si0)

        @pl.loop(0, NCHUNK // 2)
        def _(j):
            i0 = 2 * j
            start(i0 + 1, rows1, idx1, sr1, si1)
            wait(rows0, idx0, sr0, si0)
            scatter_add(rows0, idx0)
            start(i0 + 2, rows0, idx0, sr0, si0)
            wait(rows1, idx1, sr1, si1)
            scatter_add(rows1, idx1)

        wait(rows0, idx0, sr0, si0)
        scatter_add(rows0, idx0)

        plsc.subcore_barrier()
        pltpu.sync_copy(
            acc.at[pl.ds(r0, ROWS_PER_TILE)],
            out_hbm.at[cid, pl.ds(r0, ROWS_PER_TILE)],
        )

        @pl.when(sid == NS - 1)
        def _():
            pltpu.sync_copy(
                acc.at[pl.ds(NS * ROWS_PER_TILE, TAIL_ROWS)],
                out_hbm.at[cid, pl.ds(NS * ROWS_PER_TILE, TAIL_ROWS)],
            )

    return k(efeat, dst, zeros)


BR = 1000
_PREC = lax.Precision.HIGHEST


def _ln(x, g, b, eps=1e-5):
    mu = jnp.mean(x, axis=-1, keepdims=True)
    var = jnp.mean((x - mu) ** 2, axis=-1, keepdims=True)
    return (x - mu) / jnp.sqrt(var + eps) * g + b


def _silu(x):
    return x / (1.0 + jnp.exp(-x))


def _mlp_body(nf, a0, a1, fl, w1n, w1e, w1f, b1, g1, be1, w2, b2, g2, be2, w3, b3, out):
    x_n = nf[...]
    agg = a0[...] + a1[...]
    h = (
        jnp.dot(x_n, w1n[...], precision=_PREC)
        + jnp.dot(agg, w1e[...], precision=_PREC)
        + jnp.dot(fl[...], w1f[...], precision=_PREC)
        + b1[...]
    )
    h = _silu(_ln(h, g1[...], be1[...]))
    h = jnp.dot(h, w2[...], precision=_PREC) + b2[...]
    h = _silu(_ln(h, g2[...], be2[...]))
    out[...] = jnp.dot(h, w3[...], precision=_PREC) + b3[...] + x_n


def _mlp_tc(nfeat, agg0, agg1, flow, w1n, w1e, w1f, b1, g1, be1, w2, b2, g2, be2, w3, b3):
    row_block = lambda d: pl.BlockSpec((BR, d), lambda i: (i, 0))
    full = lambda s: pl.BlockSpec(s, lambda i: (0, 0))
    return pl.pallas_call(
        _mlp_body,
        grid=(N // BR,),
        in_specs=[
            row_block(D_N),
            row_block(D_E),
            row_block(D_E),
            row_block(D_F),
            full((D_N, H)),
            full((D_E, H)),
            full((D_F, H)),
            full((1, H)),
            full((1, H)),
            full((1, H)),
            full((H, H)),
            full((1, H)),
            full((1, H)),
            full((1, H)),
            full((H, D_OUT)),
            full((1, D_OUT)),
        ],
        out_specs=row_block(D_OUT),
        out_shape=jax.ShapeDtypeStruct((N, D_OUT), jnp.float32),
    )(nfeat, agg0, agg1, flow, w1n, w1e, w1f, b1, g1, be1, w2, b2, g2, be2, w3, b3)


def kernel(efeat, nfeat, flow_features, edge_index,
           W1, b1, g1, be1, W2, b2, g2, be2, W3, b3):
    dst1d = edge_index[1].astype(jnp.int32)
    zeros = jnp.zeros((N, D_E), jnp.float32)
    agg2 = _segment_sum_sc(efeat, dst1d, zeros)
    w1n = W1[:D_N]
    w1e = W1[D_N:D_N + D_E]
    w1f = W1[D_N + D_E:]
    r = lambda v: v.reshape(1, -1)
    nfeat_new = _mlp_tc(
        nfeat, agg2[0], agg2[1], flow_features,
        w1n, w1e, w1f, r(b1), r(g1), r(be1),
        W2, r(b2), r(g2), r(be2), W3, r(b3),
    )
    return (efeat, nfeat_new)


# speedup vs baseline: 4.1864x; 1.1741x over previous
import functools

import jax
import jax.numpy as jnp
from jax import lax
from jax.experimental import pallas as pl
from jax.experimental.pallas import tpu as pltpu
from jax.experimental.pallas import tpu_sc as plsc

N = 10000
E = 320000
D_N = 128
D_E = 128
D_F = 16
H = 256
D_OUT = 128

NC = 2
NS = 16
CHUNK = 80
ROWS_PER_TILE = 624
TAIL_ROWS = N - NS * ROWS_PER_TILE


EPW = E // (NC * NS)
NCHUNK = EPW // CHUNK


def _segment_sum_sc(efeat, dst, zeros):
    mesh = plsc.VectorSubcoreMesh(
        core_axis_name="c", subcore_axis_name="s", num_cores=NC, num_subcores=NS
    )

    @functools.partial(
        pl.kernel,
        out_type=jax.ShapeDtypeStruct((NC, N, D_E), jnp.float32),
        mesh=mesh,
        scratch_types=[
            pltpu.VMEM_SHARED((N, D_E), jnp.float32),
            pltpu.VMEM((CHUNK, D_E), jnp.float32),
            pltpu.VMEM((CHUNK, D_E), jnp.float32),
            pltpu.VMEM((CHUNK,), jnp.int32),
            pltpu.VMEM((CHUNK,), jnp.int32),
            pltpu.SemaphoreType.DMA,
            pltpu.SemaphoreType.DMA,
            pltpu.SemaphoreType.DMA,
            pltpu.SemaphoreType.DMA,
        ],
    )
    def k(efeat_hbm, dst_hbm, z_hbm, out_hbm, acc,
          rows0, rows1, idx0, idx1, sr0, sr1, si0, si1):
        cid = lax.axis_index("c")
        sid = lax.axis_index("s")
        r0 = sid * ROWS_PER_TILE
        pltpu.sync_copy(
            z_hbm.at[pl.ds(r0, ROWS_PER_TILE)], acc.at[pl.ds(r0, ROWS_PER_TILE)]
        )

        @pl.when(sid == NS - 1)
        def _():
            pltpu.sync_copy(
                z_hbm.at[pl.ds(NS * ROWS_PER_TILE, TAIL_ROWS)],
                acc.at[pl.ds(NS * ROWS_PER_TILE, TAIL_ROWS)],
            )

        plsc.subcore_barrier()

        eb = (cid * NS + sid) * EPW

        def start(i, rows, idx, sr, si):
            b = eb + i * CHUNK
            pltpu.async_copy(efeat_hbm.at[pl.ds(b, CHUNK)], rows, sr)
            pltpu.async_copy(dst_hbm.at[pl.ds(E + b, CHUNK)], idx, si)

        def wait(rows, idx, sr, si):
            pltpu.make_async_copy(efeat_hbm.at[pl.ds(0, CHUNK)], rows, sr).wait()
            pltpu.make_async_copy(dst_hbm.at[pl.ds(0, CHUNK)], idx, si).wait()

        def scatter_add(rows, idx):
            pltpu.sync_copy(rows, acc.at[idx], add=True)

        start(0, rows0, idx0, sr0, si0)

        @pl.loop(0, NCHUNK // 2)
        def _(j):
            i0 = 2 * j
            start(i0 + 1, rows1, idx1, sr1, si1)
            wait(rows0, idx0, sr0, si0)
            scatter_add(rows0, idx0)
            start(i0 + 2, rows0, idx0, sr0, si0)
            wait(rows1, idx1, sr1, si1)
            scatter_add(rows1, idx1)

        wait(rows0, idx0, sr0, si0)
        scatter_add(rows0, idx0)

        plsc.subcore_barrier()
        pltpu.sync_copy(
            acc.at[pl.ds(r0, ROWS_PER_TILE)],
            out_hbm.at[cid, pl.ds(r0, ROWS_PER_TILE)],
        )

        @pl.when(sid == NS - 1)
        def _():
            pltpu.sync_copy(
                acc.at[pl.ds(NS * ROWS_PER_TILE, TAIL_ROWS)],
                out_hbm.at[cid, pl.ds(NS * ROWS_PER_TILE, TAIL_ROWS)],
            )

    return k(efeat, dst, zeros)


BR = 1000
_PREC = lax.Precision.HIGHEST


def _ln(x, g, b, eps=1e-5):
    mu = jnp.mean(x, axis=-1, keepdims=True)
    var = jnp.mean((x - mu) ** 2, axis=-1, keepdims=True)
    return (x - mu) / jnp.sqrt(var + eps) * g + b


def _silu(x):
    return x / (1.0 + jnp.exp(-x))


def _mlp_body(nf, agg2, fl, w1n, w1e, w1f, b1, g1, be1, w2, b2, g2, be2, w3, b3, out):
    x_n = nf[...]
    agg = agg2[0] + agg2[1]
    h = (
        jnp.dot(x_n, w1n[...], precision=_PREC)
        + jnp.dot(agg, w1e[...], precision=_PREC)
        + jnp.dot(fl[...], w1f[...], precision=_PREC)
        + b1[...]
    )
    h = _silu(_ln(h, g1[...], be1[...]))
    h = jnp.dot(h, w2[...], precision=_PREC) + b2[...]
    h = _silu(_ln(h, g2[...], be2[...]))
    out[...] = jnp.dot(h, w3[...], precision=_PREC) + b3[...] + x_n


def _mlp_tc(nfeat, agg2, flow, W1, b1, g1, be1, w2, b2, g2, be2, w3, b3):
    row_block = lambda d: pl.BlockSpec((BR, d), lambda i: (i, 0))
    full = lambda s: pl.BlockSpec(s, lambda i: (0, 0))
    return pl.pallas_call(
        _mlp_body,
        grid=(N // BR,),
        in_specs=[
            row_block(D_N),
            pl.BlockSpec((NC, BR, D_E), lambda i: (0, i, 0)),
            row_block(D_F),
            pl.BlockSpec((D_N, H), lambda i: (0, 0)),
            pl.BlockSpec((D_E, H), lambda i: (1, 0)),
            pl.BlockSpec((D_F, H), lambda i: ((D_N + D_E) // D_F, 0)),
            full((1, H)),
            full((1, H)),
            full((1, H)),
            full((H, H)),
            full((1, H)),
            full((1, H)),
            full((1, H)),
            full((H, D_OUT)),
            full((1, D_OUT)),
        ],
        out_specs=row_block(D_OUT),
        out_shape=jax.ShapeDtypeStruct((N, D_OUT), jnp.float32),
    )(nfeat, agg2, flow, W1, W1, W1, b1, g1, be1, w2, b2, g2, be2, w3, b3)


CR = 8000


def _copy_body(src, dst):
    dst[...] = src[...]


def _copy_tc(efeat):
    return pl.pallas_call(
        _copy_body,
        grid=(E // CR,),
        in_specs=[pl.BlockSpec((CR, D_E), lambda i: (i, 0))],
        out_specs=pl.BlockSpec((CR, D_E), lambda i: (i, 0)),
        out_shape=jax.ShapeDtypeStruct((E, D_E), jnp.float32),
    )(efeat)


def kernel(efeat, nfeat, flow_features, edge_index,
           W1, b1, g1, be1, W2, b2, g2, be2, W3, b3):
    ei_flat = edge_index.reshape(2 * E).astype(jnp.int32)
    zeros = jnp.zeros((N, D_E), jnp.float32)
    agg2 = _segment_sum_sc(efeat, ei_flat, zeros)
    efeat_out = _copy_tc(efeat)
    r = lambda v: v.reshape(1, -1)
    nfeat_new = _mlp_tc(
        nfeat, agg2, flow_features,
        W1, r(b1), r(g1), r(be1),
        W2, r(b2), r(g2), r(be2), W3, r(b3),
    )
    return (efeat_out, nfeat_new)


# speedup vs baseline: 4.6283x; 1.1056x over previous
import functools

import jax
import jax.numpy as jnp
from jax import lax
from jax.experimental import pallas as pl
from jax.experimental.pallas import tpu as pltpu
from jax.experimental.pallas import tpu_sc as plsc

N = 10000
E = 320000
D_N = 128
D_E = 128
D_F = 16
H = 256
D_OUT = 128

NC = 2
NS = 16
CHUNK = 80
ROWS_PER_TILE = 624
TAIL_ROWS = N - NS * ROWS_PER_TILE


EPW = E // (NC * NS)
NCHUNK = EPW // CHUNK


def _segment_sum_sc(efeat, dst, zeros):
    mesh = plsc.VectorSubcoreMesh(
        core_axis_name="c", subcore_axis_name="s", num_cores=NC, num_subcores=NS
    )

    @functools.partial(
        pl.kernel,
        out_type=jax.ShapeDtypeStruct((NC, N, D_E), jnp.float32),
        mesh=mesh,
        scratch_types=[
            pltpu.VMEM_SHARED((N, D_E), jnp.float32),
            pltpu.VMEM((CHUNK, D_E), jnp.float32),
            pltpu.VMEM((CHUNK, D_E), jnp.float32),
            pltpu.VMEM((CHUNK,), jnp.int32),
            pltpu.VMEM((CHUNK,), jnp.int32),
            pltpu.SemaphoreType.DMA,
            pltpu.SemaphoreType.DMA,
            pltpu.SemaphoreType.DMA,
            pltpu.SemaphoreType.DMA,
        ],
    )
    def k(efeat_hbm, dst_hbm, z_hbm, out_hbm, acc,
          rows0, rows1, idx0, idx1, sr0, sr1, si0, si1):
        cid = lax.axis_index("c")
        sid = lax.axis_index("s")
        r0 = sid * ROWS_PER_TILE
        pltpu.sync_copy(
            z_hbm.at[pl.ds(r0, ROWS_PER_TILE)], acc.at[pl.ds(r0, ROWS_PER_TILE)]
        )

        @pl.when(sid == NS - 1)
        def _():
            pltpu.sync_copy(
                z_hbm.at[pl.ds(NS * ROWS_PER_TILE, TAIL_ROWS)],
                acc.at[pl.ds(NS * ROWS_PER_TILE, TAIL_ROWS)],
            )

        plsc.subcore_barrier()

        eb = (cid * NS + sid) * EPW

        def start(i, rows, idx, sr, si):
            b = eb + i * CHUNK
            pltpu.async_copy(efeat_hbm.at[pl.ds(b, CHUNK)], rows, sr)
            pltpu.async_copy(dst_hbm.at[pl.ds(E + b, CHUNK)], idx, si)

        def wait(rows, idx, sr, si):
            pltpu.make_async_copy(efeat_hbm.at[pl.ds(0, CHUNK)], rows, sr).wait()
            pltpu.make_async_copy(dst_hbm.at[pl.ds(0, CHUNK)], idx, si).wait()

        def scatter_add(rows, idx):
            pltpu.sync_copy(rows, acc.at[idx], add=True)

        start(0, rows0, idx0, sr0, si0)

        @pl.loop(0, NCHUNK // 2)
        def _(j):
            i0 = 2 * j
            start(i0 + 1, rows1, idx1, sr1, si1)
            wait(rows0, idx0, sr0, si0)
            scatter_add(rows0, idx0)
            start(i0 + 2, rows0, idx0, sr0, si0)
            wait(rows1, idx1, sr1, si1)
            scatter_add(rows1, idx1)

        wait(rows0, idx0, sr0, si0)
        scatter_add(rows0, idx0)

        plsc.subcore_barrier()
        pltpu.sync_copy(
            acc.at[pl.ds(r0, ROWS_PER_TILE)],
            out_hbm.at[cid, pl.ds(r0, ROWS_PER_TILE)],
        )

        @pl.when(sid == NS - 1)
        def _():
            pltpu.sync_copy(
                acc.at[pl.ds(NS * ROWS_PER_TILE, TAIL_ROWS)],
                out_hbm.at[cid, pl.ds(NS * ROWS_PER_TILE, TAIL_ROWS)],
            )

    return k(efeat, dst, zeros)


BR = 400
CR = E // (N // BR)
_PREC = lax.Precision.HIGHEST


def _ln(x, g, b, eps=1e-5):
    mu = jnp.mean(x, axis=-1, keepdims=True)
    var = jnp.mean((x - mu) ** 2, axis=-1, keepdims=True)
    return (x - mu) / jnp.sqrt(var + eps) * g + b


def _silu(x):
    return x / (1.0 + jnp.exp(-x))


def _mlp_body(nf, agg2, fl, ef, w1n, w1e, w1f, b1, g1, be1, w2, b2, g2, be2, w3, b3,
              out, ef_out):
    ef_out[...] = ef[...]
    x_n = nf[...]
    agg = agg2[0] + agg2[1]
    h = (
        jnp.dot(x_n, w1n[...], precision=_PREC)
        + jnp.dot(agg, w1e[...], precision=_PREC)
        + jnp.dot(fl[...], w1f[...], precision=_PREC)
        + b1[...]
    )
    h = _silu(_ln(h, g1[...], be1[...]))
    h = jnp.dot(h, w2[...], precision=_PREC) + b2[...]
    h = _silu(_ln(h, g2[...], be2[...]))
    out[...] = jnp.dot(h, w3[...], precision=_PREC) + b3[...] + x_n


def _mlp_tc(nfeat, agg2, flow, efeat, W1, b1, g1, be1, w2, b2, g2, be2, w3, b3):
    row_block = lambda d: pl.BlockSpec((BR, d), lambda i: (i, 0))
    full = lambda s: pl.BlockSpec(s, lambda i: (0, 0))
    return pl.pallas_call(
        _mlp_body,
        grid=(N // BR,),
        in_specs=[
            row_block(D_N),
            pl.BlockSpec((NC, BR, D_E), lambda i: (0, i, 0)),
            row_block(D_F),
            pl.BlockSpec((CR, D_E), lambda i: (i, 0)),
            pl.BlockSpec((D_N, H), lambda i: (0, 0)),
            pl.BlockSpec((D_E, H), lambda i: (1, 0)),
            pl.BlockSpec((D_F, H), lambda i: ((D_N + D_E) // D_F, 0)),
            full((1, H)),
            full((1, H)),
            full((1, H)),
            full((H, H)),
            full((1, H)),
            full((1, H)),
            full((1, H)),
            full((H, D_OUT)),
            full((1, D_OUT)),
        ],
        out_specs=[
            row_block(D_OUT),
            pl.BlockSpec((CR, D_E), lambda i: (i, 0)),
        ],
        out_shape=[
            jax.ShapeDtypeStruct((N, D_OUT), jnp.float32),
            jax.ShapeDtypeStruct((E, D_E), jnp.float32),
        ],
    )(nfeat, agg2, flow, efeat, W1, W1, W1, b1, g1, be1, w2, b2, g2, be2, w3, b3)


def kernel(efeat, nfeat, flow_features, edge_index,
           W1, b1, g1, be1, W2, b2, g2, be2, W3, b3):
    ei_flat = edge_index.reshape(2 * E).astype(jnp.int32)
    zeros = jnp.zeros((N, D_E), jnp.float32)
    agg2 = _segment_sum_sc(efeat, ei_flat, zeros)
    r = lambda v: v.reshape(1, -1)
    nfeat_new, efeat_out = _mlp_tc(
        nfeat, agg2, flow_features, efeat,
        W1, r(b1), r(g1), r(be1),
        W2, r(b2), r(g2), r(be2), W3, r(b3),
    )
    return (efeat_out, nfeat_new)


# speedup vs baseline: 4.9236x; 1.0638x over previous
import functools

import jax
import jax.numpy as jnp
from jax import lax
from jax.experimental import pallas as pl
from jax.experimental.pallas import tpu as pltpu
from jax.experimental.pallas import tpu_sc as plsc

N = 10000
E = 320000
D_N = 128
D_E = 128
D_F = 16
H = 256
D_OUT = 128

NC = 2
NS = 16
CHUNK = 80
ROWS_PER_TILE = 624
TAIL_ROWS = N - NS * ROWS_PER_TILE


EPW = E // (NC * NS)
NCHUNK = EPW // CHUNK


def _segment_sum_sc(efeat, dst, zeros):
    mesh = plsc.VectorSubcoreMesh(
        core_axis_name="c", subcore_axis_name="s", num_cores=NC, num_subcores=NS
    )

    @functools.partial(
        pl.kernel,
        out_type=(
            jax.ShapeDtypeStruct((NC, N, D_E), jnp.float32),
            jax.ShapeDtypeStruct((E, D_E), jnp.float32),
        ),
        mesh=mesh,
        scratch_types=[
            pltpu.VMEM_SHARED((N, D_E), jnp.float32),
            pltpu.VMEM((CHUNK, D_E), jnp.float32),
            pltpu.VMEM((CHUNK, D_E), jnp.float32),
            pltpu.VMEM((CHUNK,), jnp.int32),
            pltpu.VMEM((CHUNK,), jnp.int32),
            pltpu.SemaphoreType.DMA,
            pltpu.SemaphoreType.DMA,
            pltpu.SemaphoreType.DMA,
            pltpu.SemaphoreType.DMA,
            pltpu.SemaphoreType.DMA,
            pltpu.SemaphoreType.DMA,
        ],
    )
    def k(efeat_hbm, dst_hbm, z_hbm, out_hbm, eout_hbm, acc,
          rows0, rows1, idx0, idx1, sr0, sr1, si0, si1, sw0, sw1):
        cid = lax.axis_index("c")
        sid = lax.axis_index("s")
        r0 = sid * ROWS_PER_TILE
        pltpu.sync_copy(
            z_hbm.at[pl.ds(r0, ROWS_PER_TILE)], acc.at[pl.ds(r0, ROWS_PER_TILE)]
        )

        @pl.when(sid == NS - 1)
        def _():
            pltpu.sync_copy(
                z_hbm.at[pl.ds(NS * ROWS_PER_TILE, TAIL_ROWS)],
                acc.at[pl.ds(NS * ROWS_PER_TILE, TAIL_ROWS)],
            )

        plsc.subcore_barrier()

        eb = (cid * NS + sid) * EPW

        def start(i, rows, idx, sr, si):
            b = eb + i * CHUNK
            pltpu.async_copy(efeat_hbm.at[pl.ds(b, CHUNK)], rows, sr)
            pltpu.async_copy(dst_hbm.at[pl.ds(E + b, CHUNK)], idx, si)

        def wait(rows, idx, sr, si):
            pltpu.make_async_copy(efeat_hbm.at[pl.ds(0, CHUNK)], rows, sr).wait()
            pltpu.make_async_copy(dst_hbm.at[pl.ds(0, CHUNK)], idx, si).wait()

        def scatter_add(rows, idx):
            pltpu.sync_copy(rows, acc.at[idx], add=True)

        def wb_start(i, rows, sw):
            pltpu.async_copy(rows, eout_hbm.at[pl.ds(eb + i * CHUNK, CHUNK)], sw)

        def wb_wait(rows, sw):
            pltpu.make_async_copy(rows, eout_hbm.at[pl.ds(0, CHUNK)], sw).wait()

        start(0, rows0, idx0, sr0, si0)

        @pl.loop(0, NCHUNK // 2)
        def _(j):
            i0 = 2 * j

            @pl.when(j > 0)
            def _():
                wb_wait(rows1, sw1)

            start(i0 + 1, rows1, idx1, sr1, si1)
            wait(rows0, idx0, sr0, si0)
            scatter_add(rows0, idx0)
            wb_start(i0, rows0, sw0)
            wait(rows1, idx1, sr1, si1)
            scatter_add(rows1, idx1)
            wb_start(i0 + 1, rows1, sw1)
            wb_wait(rows0, sw0)
            start(i0 + 2, rows0, idx0, sr0, si0)

        wb_wait(rows1, sw1)
        wait(rows0, idx0, sr0, si0)
        scatter_add(rows0, idx0)
        wb_start(NCHUNK - 1, rows0, sw0)
        wb_wait(rows0, sw0)

        plsc.subcore_barrier()
        pltpu.sync_copy(
            acc.at[pl.ds(r0, ROWS_PER_TILE)],
            out_hbm.at[cid, pl.ds(r0, ROWS_PER_TILE)],
        )

        @pl.when(sid == NS - 1)
        def _():
            pltpu.sync_copy(
                acc.at[pl.ds(NS * ROWS_PER_TILE, TAIL_ROWS)],
                out_hbm.at[cid, pl.ds(NS * ROWS_PER_TILE, TAIL_ROWS)],
            )

    return k(efeat, dst, zeros)


BR = 1000


def _ln(x, g, b, eps=1e-5):
    mu = jnp.mean(x, axis=-1, keepdims=True)
    var = jnp.mean((x - mu) ** 2, axis=-1, keepdims=True)
    return (x - mu) / jnp.sqrt(var + eps) * g + b


def _silu(x):
    return x / (1.0 + jnp.exp(-x))


def _dot3(x, w):
    xh = x.astype(jnp.bfloat16)
    xl = (x - xh.astype(jnp.float32)).astype(jnp.bfloat16)
    wh = w.astype(jnp.bfloat16)
    wl = (w - wh.astype(jnp.float32)).astype(jnp.bfloat16)
    d = lambda a, b: jax.lax.dot_general(
        a, b, (((1,), (0,)), ((), ())), preferred_element_type=jnp.float32
    )
    return d(xh, wh) + d(xh, wl) + d(xl, wh)


def _mlp_body(nf, agg2, fl, w1n, w1e, w1f, b1, g1, be1, w2, b2, g2, be2, w3, b3, out):
    x_n = nf[...]
    agg = agg2[0] + agg2[1]
    h = (
        _dot3(x_n, w1n[...])
        + _dot3(agg, w1e[...])
        + _dot3(fl[...], w1f[...])
        + b1[...]
    )
    h = _silu(_ln(h, g1[...], be1[...]))
    h = _dot3(h, w2[...]) + b2[...]
    h = _silu(_ln(h, g2[...], be2[...]))
    out[...] = _dot3(h, w3[...]) + b3[...] + x_n


def _mlp_tc(nfeat, agg2, flow, W1, b1, g1, be1, w2, b2, g2, be2, w3, b3):
    row_block = lambda d: pl.BlockSpec((BR, d), lambda i: (i, 0))
    full = lambda s: pl.BlockSpec(s, lambda i: (0, 0))
    return pl.pallas_call(
        _mlp_body,
        grid=(N // BR,),
        in_specs=[
            row_block(D_N),
            pl.BlockSpec((NC, BR, D_E), lambda i: (0, i, 0)),
            row_block(D_F),
            pl.BlockSpec((D_N, H), lambda i: (0, 0)),
            pl.BlockSpec((D_E, H), lambda i: (1, 0)),
            pl.BlockSpec((D_F, H), lambda i: ((D_N + D_E) // D_F, 0)),
            full((1, H)),
            full((1, H)),
            full((1, H)),
            full((H, H)),
            full((1, H)),
            full((1, H)),
            full((1, H)),
            full((H, D_OUT)),
            full((1, D_OUT)),
        ],
        out_specs=row_block(D_OUT),
        out_shape=jax.ShapeDtypeStruct((N, D_OUT), jnp.float32),
    )(nfeat, agg2, flow, W1, W1, W1, b1, g1, be1, w2, b2, g2, be2, w3, b3)


def kernel(efeat, nfeat, flow_features, edge_index,
           W1, b1, g1, be1, W2, b2, g2, be2, W3, b3):
    ei_flat = edge_index.reshape(2 * E).astype(jnp.int32)
    zeros = jnp.zeros((N, D_E), jnp.float32)
    agg2, efeat_out = _segment_sum_sc(efeat, ei_flat, zeros)
    r = lambda v: v.reshape(1, -1)
    nfeat_new = _mlp_tc(
        nfeat, agg2, flow_features,
        W1, r(b1), r(g1), r(be1),
        W2, r(b2), r(g2), r(be2), W3, r(b3),
    )
    return (efeat_out, nfeat_new)


# speedup vs baseline: 5.0092x; 1.0174x over previous
import functools

import jax
import jax.numpy as jnp
from jax import lax
from jax.experimental import pallas as pl
from jax.experimental.pallas import tpu as pltpu
from jax.experimental.pallas import tpu_sc as plsc

N = 10000
E = 320000
D_N = 128
D_E = 128
D_F = 16
H = 256
D_OUT = 128

NC = 2
NS = 16
CHUNK = 80
ROWS_PER_TILE = 624
TAIL_ROWS = N - NS * ROWS_PER_TILE


EPW = E // (NC * NS)
NCHUNK = EPW // CHUNK


def _segment_sum_sc(efeat, dst, zeros):
    mesh = plsc.VectorSubcoreMesh(
        core_axis_name="c", subcore_axis_name="s", num_cores=NC, num_subcores=NS
    )

    @functools.partial(
        pl.kernel,
        out_type=(
            jax.ShapeDtypeStruct((NC, N, D_E), jnp.float32),
            jax.ShapeDtypeStruct((E, D_E), jnp.float32),
        ),
        mesh=mesh,
        scratch_types=[
            pltpu.VMEM_SHARED((N, D_E), jnp.float32),
            pltpu.VMEM((CHUNK, D_E), jnp.float32),
            pltpu.VMEM((CHUNK, D_E), jnp.float32),
            pltpu.VMEM((CHUNK,), jnp.int32),
            pltpu.VMEM((CHUNK,), jnp.int32),
            pltpu.SemaphoreType.DMA,
            pltpu.SemaphoreType.DMA,
            pltpu.SemaphoreType.DMA,
            pltpu.SemaphoreType.DMA,
            pltpu.SemaphoreType.DMA,
            pltpu.SemaphoreType.DMA,
            pltpu.SemaphoreType.DMA,
            pltpu.SemaphoreType.DMA,
        ],
    )
    def k(efeat_hbm, dst_hbm, z_hbm, out_hbm, eout_hbm, acc,
          rows0, rows1, idx0, idx1, sr0, sr1, si0, si1, sw0, sw1, ss0, ss1):
        cid = lax.axis_index("c")
        sid = lax.axis_index("s")
        r0 = sid * ROWS_PER_TILE
        pltpu.sync_copy(
            z_hbm.at[pl.ds(r0, ROWS_PER_TILE)], acc.at[pl.ds(r0, ROWS_PER_TILE)]
        )

        @pl.when(sid == NS - 1)
        def _():
            pltpu.sync_copy(
                z_hbm.at[pl.ds(NS * ROWS_PER_TILE, TAIL_ROWS)],
                acc.at[pl.ds(NS * ROWS_PER_TILE, TAIL_ROWS)],
            )

        plsc.subcore_barrier()

        eb = (cid * NS + sid) * EPW

        def start(i, rows, idx, sr, si):
            b = eb + i * CHUNK
            pltpu.async_copy(efeat_hbm.at[pl.ds(b, CHUNK)], rows, sr)
            pltpu.async_copy(dst_hbm.at[pl.ds(E + b, CHUNK)], idx, si)

        def wait(rows, idx, sr, si):
            pltpu.make_async_copy(efeat_hbm.at[pl.ds(0, CHUNK)], rows, sr).wait()
            pltpu.make_async_copy(dst_hbm.at[pl.ds(0, CHUNK)], idx, si).wait()

        def sc_start(rows, idx, ss):
            pltpu.async_copy(rows, acc.at[idx], ss, add=True)

        def sc_wait(rows, idx, ss):
            pltpu.make_async_copy(rows, acc.at[idx], ss).wait()

        def wb_start(i, rows, sw):
            pltpu.async_copy(rows, eout_hbm.at[pl.ds(eb + i * CHUNK, CHUNK)], sw)

        def wb_wait(rows, sw):
            pltpu.make_async_copy(rows, eout_hbm.at[pl.ds(0, CHUNK)], sw).wait()

        start(0, rows0, idx0, sr0, si0)

        @pl.loop(0, NCHUNK // 2)
        def _(j):
            i0 = 2 * j

            @pl.when(j > 0)
            def _():
                sc_wait(rows1, idx1, ss1)
                wb_wait(rows1, sw1)

            start(i0 + 1, rows1, idx1, sr1, si1)
            wait(rows0, idx0, sr0, si0)
            sc_start(rows0, idx0, ss0)
            wb_start(i0, rows0, sw0)
            wait(rows1, idx1, sr1, si1)
            sc_start(rows1, idx1, ss1)
            wb_start(i0 + 1, rows1, sw1)
            sc_wait(rows0, idx0, ss0)
            wb_wait(rows0, sw0)
            start(i0 + 2, rows0, idx0, sr0, si0)

        sc_wait(rows1, idx1, ss1)
        wb_wait(rows1, sw1)
        wait(rows0, idx0, sr0, si0)
        sc_start(rows0, idx0, ss0)
        wb_start(NCHUNK - 1, rows0, sw0)
        sc_wait(rows0, idx0, ss0)
        wb_wait(rows0, sw0)

        plsc.subcore_barrier()
        pltpu.sync_copy(
            acc.at[pl.ds(r0, ROWS_PER_TILE)],
            out_hbm.at[cid, pl.ds(r0, ROWS_PER_TILE)],
        )

        @pl.when(sid == NS - 1)
        def _():
            pltpu.sync_copy(
                acc.at[pl.ds(NS * ROWS_PER_TILE, TAIL_ROWS)],
                out_hbm.at[cid, pl.ds(NS * ROWS_PER_TILE, TAIL_ROWS)],
            )

    return k(efeat, dst, zeros)


BR = 1000


def _ln(x, g, b, eps=1e-5):
    mu = jnp.mean(x, axis=-1, keepdims=True)
    var = jnp.mean((x - mu) ** 2, axis=-1, keepdims=True)
    return (x - mu) / jnp.sqrt(var + eps) * g + b


def _silu(x):
    return x / (1.0 + jnp.exp(-x))


def _dot3(x, wh, wl):
    xh = x.astype(jnp.bfloat16)
    xl = (x - xh.astype(jnp.float32)).astype(jnp.bfloat16)
    d = lambda a, b: jax.lax.dot_general(
        a, b, (((1,), (0,)), ((), ())), preferred_element_type=jnp.float32
    )
    return d(xh, wh) + d(xh, wl) + d(xl, wh)


def _mlp_body(nf, agg2, fl, w1nh, w1nl, w1eh, w1el, w1fh, w1fl, b1, g1, be1,
              w2h, w2l, b2, g2, be2, w3h, w3l, b3, out):
    x_n = nf[...]
    agg = agg2[0] + agg2[1]
    h = (
        _dot3(x_n, w1nh[...], w1nl[...])
        + _dot3(agg, w1eh[...], w1el[...])
        + _dot3(fl[...], w1fh[...], w1fl[...])
        + b1[...]
    )
    h = _silu(_ln(h, g1[...], be1[...]))
    h = _dot3(h, w2h[...], w2l[...]) + b2[...]
    h = _silu(_ln(h, g2[...], be2[...]))
    out[...] = _dot3(h, w3h[...], w3l[...]) + b3[...] + x_n


def _mlp_tc(nfeat, agg2, flow, w1h, w1l, b1, g1, be1, w2h, w2l, b2, g2, be2,
            w3h, w3l, b3):
    row_block = lambda d: pl.BlockSpec((BR, d), lambda i: (i, 0))
    full = lambda s: pl.BlockSpec(s, lambda i: (0, 0))
    w1_slabs = [
        pl.BlockSpec((D_N, H), lambda i: (0, 0)),
        pl.BlockSpec((D_E, H), lambda i: (1, 0)),
        pl.BlockSpec((D_F, H), lambda i: ((D_N + D_E) // D_F, 0)),
    ]
    return pl.pallas_call(
        _mlp_body,
        grid=(N // BR,),
        in_specs=[
            row_block(D_N),
            pl.BlockSpec((NC, BR, D_E), lambda i: (0, i, 0)),
            row_block(D_F),
            w1_slabs[0], w1_slabs[0],
            w1_slabs[1], w1_slabs[1],
            w1_slabs[2], w1_slabs[2],
            full((1, H)),
            full((1, H)),
            full((1, H)),
            full((H, H)), full((H, H)),
            full((1, H)),
            full((1, H)),
            full((1, H)),
            full((H, D_OUT)), full((H, D_OUT)),
            full((1, D_OUT)),
        ],
        out_specs=row_block(D_OUT),
        out_shape=jax.ShapeDtypeStruct((N, D_OUT), jnp.float32),
    )(nfeat, agg2, flow, w1h, w1l, w1h, w1l, w1h, w1l, b1, g1, be1,
      w2h, w2l, b2, g2, be2, w3h, w3l, b3)


def _split_bf16(w):
    wh = w.astype(jnp.bfloat16)
    wl = (w - wh.astype(jnp.float32)).astype(jnp.bfloat16)
    return wh, wl


def kernel(efeat, nfeat, flow_features, edge_index,
           W1, b1, g1, be1, W2, b2, g2, be2, W3, b3):
    ei_flat = edge_index.reshape(2 * E).astype(jnp.int32)
    zeros = jnp.zeros((N, D_E), jnp.float32)
    agg2, efeat_out = _segment_sum_sc(efeat, ei_flat, zeros)
    w1h, w1l = _split_bf16(W1)
    w2h, w2l = _split_bf16(W2)
    w3h, w3l = _split_bf16(W3)
    r = lambda v: v.reshape(1, -1)
    nfeat_new = _mlp_tc(
        nfeat, agg2, flow_features,
        w1h, w1l, r(b1), r(g1), r(be1),
        w2h, w2l, r(b2), r(g2), r(be2), w3h, w3l, r(b3),
    )
    return (efeat_out, nfeat_new)


# speedup vs baseline: 5.3461x; 1.0673x over previous
import functools

import jax
import jax.numpy as jnp
from jax import lax
from jax.experimental import pallas as pl
from jax.experimental.pallas import tpu as pltpu
from jax.experimental.pallas import tpu_sc as plsc

N = 10000
E = 320000
D_N = 128
D_E = 128
D_F = 16
H = 256
D_OUT = 128

NC = 2
NS = 16
CHUNK = 128
ROWS_PER_TILE = 624
TAIL_ROWS = N - NS * ROWS_PER_TILE

NCHUNKS = E // CHUNK
CHUNK_BASE = NCHUNKS // (NC * NS)
CHUNK_EXTRA = NCHUNKS - CHUNK_BASE * NC * NS
ZROWS = 640


def _segment_sum_sc(efeat, edge_index, zeros):
    mesh = plsc.VectorSubcoreMesh(
        core_axis_name="c", subcore_axis_name="s", num_cores=NC, num_subcores=NS
    )

    @functools.partial(
        pl.kernel,
        out_type=(
            jax.ShapeDtypeStruct((NC, N, D_E), jnp.float32),
            jax.ShapeDtypeStruct((E, D_E), jnp.float32),
        ),
        mesh=mesh,
        scratch_types=[
            pltpu.VMEM_SHARED((N, D_E), jnp.float32),
            pltpu.VMEM((CHUNK, D_E), jnp.float32),
            pltpu.VMEM((CHUNK, D_E), jnp.float32),
            pltpu.VMEM((2, CHUNK), jnp.int32),
            pltpu.VMEM((2, CHUNK), jnp.int32),
            pltpu.SemaphoreType.DMA,
            pltpu.SemaphoreType.DMA,
            pltpu.SemaphoreType.DMA,
            pltpu.SemaphoreType.DMA,
            pltpu.SemaphoreType.DMA,
            pltpu.SemaphoreType.DMA,
            pltpu.SemaphoreType.DMA,
            pltpu.SemaphoreType.DMA,
        ],
    )
    def k(efeat_hbm, ei_hbm, z_hbm, out_hbm, eout_hbm, acc,
          rows0, rows1, idx0, idx1, sr0, sr1, si0, si1, sw0, sw1, ss0, ss1):
        cid = lax.axis_index("c")
        sid = lax.axis_index("s")
        w = cid * NS + sid
        r0 = sid * ROWS_PER_TILE
        pltpu.sync_copy(
            z_hbm.at[pl.ds(0, ROWS_PER_TILE)], acc.at[pl.ds(r0, ROWS_PER_TILE)]
        )

        @pl.when(sid == NS - 1)
        def _():
            pltpu.sync_copy(
                z_hbm.at[pl.ds(0, TAIL_ROWS)],
                acc.at[pl.ds(NS * ROWS_PER_TILE, TAIL_ROWS)],
            )

        nc_w = CHUNK_BASE + jnp.where(w < CHUNK_EXTRA, 1, 0)
        eb = (CHUNK_BASE * w + jnp.minimum(w, CHUNK_EXTRA)) * CHUNK

        def start(i, rows, idx, sr, si):
            b = eb + i * CHUNK
            pltpu.async_copy(efeat_hbm.at[pl.ds(b, CHUNK)], rows, sr)
            pltpu.async_copy(ei_hbm.at[pl.ds(0, 2), pl.ds(b, CHUNK)], idx, si)

        def wait(rows, idx, sr, si):
            pltpu.make_async_copy(efeat_hbm.at[pl.ds(0, CHUNK)], rows, sr).wait()
            pltpu.make_async_copy(
                ei_hbm.at[pl.ds(0, 2), pl.ds(0, CHUNK)], idx, si
            ).wait()

        def sc_start(rows, idx, ss):
            pltpu.async_copy(rows, acc.at[idx.at[1]], ss, add=True)

        def sc_wait(rows, idx, ss):
            pltpu.make_async_copy(rows, acc.at[idx.at[1]], ss).wait()

        def wb_start(i, rows, sw):
            pltpu.async_copy(rows, eout_hbm.at[pl.ds(eb + i * CHUNK, CHUNK)], sw)

        def wb_wait(rows, sw):
            pltpu.make_async_copy(rows, eout_hbm.at[pl.ds(0, CHUNK)], sw).wait()

        start(0, rows0, idx0, sr0, si0)
        plsc.subcore_barrier()

        @pl.loop(0, nc_w // 2)
        def _(j):
            i0 = 2 * j

            @pl.when(j > 0)
            def _():
                sc_wait(rows1, idx1, ss1)
                wb_wait(rows1, sw1)

            start(i0 + 1, rows1, idx1, sr1, si1)
            wait(rows0, idx0, sr0, si0)
            sc_start(rows0, idx0, ss0)
            wb_start(i0, rows0, sw0)
            wait(rows1, idx1, sr1, si1)
            sc_start(rows1, idx1, ss1)
            wb_start(i0 + 1, rows1, sw1)
            sc_wait(rows0, idx0, ss0)
            wb_wait(rows0, sw0)

            @pl.when(i0 + 2 < nc_w)
            def _():
                start(i0 + 2, rows0, idx0, sr0, si0)

        sc_wait(rows1, idx1, ss1)
        wb_wait(rows1, sw1)

        @pl.when(nc_w % 2 == 1)
        def _():
            wait(rows0, idx0, sr0, si0)
            sc_start(rows0, idx0, ss0)
            wb_start(nc_w - 1, rows0, sw0)
            sc_wait(rows0, idx0, ss0)
            wb_wait(rows0, sw0)

        plsc.subcore_barrier()
        pltpu.sync_copy(
            acc.at[pl.ds(r0, ROWS_PER_TILE)],
            out_hbm.at[cid, pl.ds(r0, ROWS_PER_TILE)],
        )

        @pl.when(sid == NS - 1)
        def _():
            pltpu.sync_copy(
                acc.at[pl.ds(NS * ROWS_PER_TILE, TAIL_ROWS)],
                out_hbm.at[cid, pl.ds(NS * ROWS_PER_TILE, TAIL_ROWS)],
            )

    return k(efeat, edge_index, zeros)


BR = 1000


def _ln(x, g, b, eps=1e-5):
    mu = jnp.mean(x, axis=-1, keepdims=True)
    var = jnp.mean((x - mu) ** 2, axis=-1, keepdims=True)
    return (x - mu) / jnp.sqrt(var + eps) * g + b


def _silu(x):
    return x / (1.0 + jnp.exp(-x))


def _dot3(x, wh, wl):
    xh = x.astype(jnp.bfloat16)
    xl = (x - xh.astype(jnp.float32)).astype(jnp.bfloat16)
    d = lambda a, b: jax.lax.dot_general(
        a, b, (((1,), (0,)), ((), ())), preferred_element_type=jnp.float32
    )
    return d(xh, wh) + d(xh, wl) + d(xl, wh)


def _mlp_body(nf, agg2, fl, w1nh, w1nl, w1eh, w1el, w1fh, w1fl, b1, g1, be1,
              w2h, w2l, b2, g2, be2, w3h, w3l, b3, out):
    x_n = nf[...]
    agg = agg2[0] + agg2[1]
    h = (
        _dot3(x_n, w1nh[...], w1nl[...])
        + _dot3(agg, w1eh[...], w1el[...])
        + _dot3(fl[...], w1fh[...], w1fl[...])
        + b1[...]
    )
    h = _silu(_ln(h, g1[...], be1[...]))
    h = _dot3(h, w2h[...], w2l[...]) + b2[...]
    h = _silu(_ln(h, g2[...], be2[...]))
    out[...] = _dot3(h, w3h[...], w3l[...]) + b3[...] + x_n


def _mlp_tc(nfeat, agg2, flow, w1h, w1l, b1, g1, be1, w2h, w2l, b2, g2, be2,
            w3h, w3l, b3):
    row_block = lambda d: pl.BlockSpec((BR, d), lambda i: (i, 0))
    full = lambda s: pl.BlockSpec(s, lambda i: (0, 0))
    w1_slabs = [
        pl.BlockSpec((D_N, H), lambda i: (0, 0)),
        pl.BlockSpec((D_E, H), lambda i: (1, 0)),
        pl.BlockSpec((D_F, H), lambda i: ((D_N + D_E) // D_F, 0)),
    ]
    return pl.pallas_call(
        _mlp_body,
        grid=(N // BR,),
        in_specs=[
            row_block(D_N),
            pl.BlockSpec((NC, BR, D_E), lambda i: (0, i, 0)),
            row_block(D_F),
            w1_slabs[0], w1_slabs[0],
            w1_slabs[1], w1_slabs[1],
            w1_slabs[2], w1_slabs[2],
            full((1, H)),
            full((1, H)),
            full((1, H)),
            full((H, H)), full((H, H)),
            full((1, H)),
            full((1, H)),
            full((1, H)),
            full((H, D_OUT)), full((H, D_OUT)),
            full((1, D_OUT)),
        ],
        out_specs=row_block(D_OUT),
        out_shape=jax.ShapeDtypeStruct((N, D_OUT), jnp.float32),
    )(nfeat, agg2, flow, w1h, w1l, w1h, w1l, w1h, w1l, b1, g1, be1,
      w2h, w2l, b2, g2, be2, w3h, w3l, b3)


def _split_bf16(w):
    wh = w.astype(jnp.bfloat16)
    wl = (w - wh.astype(jnp.float32)).astype(jnp.bfloat16)
    return wh, wl


def kernel(efeat, nfeat, flow_features, edge_index,
           W1, b1, g1, be1, W2, b2, g2, be2, W3, b3):
    zeros = jnp.zeros((ZROWS, D_E), jnp.float32)
    agg2, efeat_out = _segment_sum_sc(efeat, edge_index.astype(jnp.int32), zeros)
    w1h, w1l = _split_bf16(W1)
    w2h, w2l = _split_bf16(W2)
    w3h, w3l = _split_bf16(W3)
    r = lambda v: v.reshape(1, -1)
    nfeat_new = _mlp_tc(
        nfeat, agg2, flow_features,
        w1h, w1l, r(b1), r(g1), r(be1),
        w2h, w2l, r(b2), r(g2), r(be2), w3h, w3l, r(b3),
    )
    return (efeat_out, nfeat_new)
